# baseline (speedup 1.0000x reference)
import functools

import jax
import jax.numpy as jnp
from jax import lax
from jax.experimental import pallas as pl
from jax.experimental.pallas import tpu as pltpu
from jax.experimental.pallas import tpu_sc as plsc

B = 4
C3 = 30
C2 = 10
H = 512
W = 512
HW = H * W

NC = 2
NS = 16
L = 16
NW = NC * NS

PIX_PER_W = (B * HW) // NW
CHUNK = 2048
CHUNKS_PER_B = (HW // NW) // CHUNK

_mesh = plsc.VectorSubcoreMesh(core_axis_name="c", subcore_axis_name="s")


@functools.partial(
    pl.kernel,
    mesh=_mesh,
    out_type=jax.ShapeDtypeStruct((NW, L), jnp.float32),
    scratch_types=[
        pltpu.VMEM((C3, CHUNK), jnp.float32),
        pltpu.VMEM((C2, CHUNK), jnp.float32),
        pltpu.VMEM((32,), jnp.int32),
        pltpu.VMEM((L,), jnp.float32),
        pltpu.VMEM((L,), jnp.float32),
    ],
)
def _sc_loss(l2_hbm, l3_hbm, map_hbm, w_hbm, out_hbm,
             l3v, l2v, mapv, wv, ov):
    wid = lax.axis_index("s") * NC + lax.axis_index("c")

    pltpu.sync_copy(map_hbm, mapv)
    pltpu.sync_copy(w_hbm, wv)

    mv0 = mapv[pl.ds(0, L)]
    mv1 = mapv[pl.ds(L, L)]
    mks = [mv0[k] if k < L else mv1[k - L] for k in range(C3)]

    def chunk_step(t, acc):
        b = t // CHUNKS_PER_B
        c = t % CHUNKS_PER_B
        off = wid * (HW // NW) + c * CHUNK
        pltpu.sync_copy(l3_hbm.at[b, :, pl.ds(off, CHUNK)], l3v)
        pltpu.sync_copy(l2_hbm.at[b, :, pl.ds(off, CHUNK)], l2v)

        def group(j, acc_in):
            s = pl.ds(j * L, L)
            m3 = l3v[0, s]
            mapped = jnp.full((L,), mks[0], jnp.int32)
            for k in range(1, C3):
                v = l3v[k, s]
                upd = v > m3
                m3 = jnp.where(upd, v, m3)
                mapped = jnp.where(upd, mks[k], mapped)
            m2 = l2v[0, s]
            i2 = jnp.zeros((L,), jnp.int32)
            for k in range(1, C2):
                v = l2v[k, s]
                upd = v > m2
                m2 = jnp.where(upd, v, m2)
                i2 = jnp.where(upd, k, i2)
            return acc_in + jnp.where(mapped != i2, 1.0, 0.0)

        return lax.fori_loop(0, CHUNK // L, group, acc)

    acc = lax.fori_loop(0, B * CHUNKS_PER_B, chunk_step,
                        jnp.zeros((L,), jnp.float32))
    ov[...] = acc * wv[...] * (1.0 / (B * HW))
    pltpu.sync_copy(ov, out_hbm.at[wid])


@jax.jit
def _loss(level2_pred, level3_pred, mapping, weight):
    l2 = level2_pred.reshape(B, C2, HW)
    l3 = level3_pred.reshape(B, C3, HW)
    map_pad = jnp.pad(mapping, (0, 32 - C3))
    w_vec = jnp.broadcast_to(jnp.asarray(weight, jnp.float32), (L,))
    partials = _sc_loss(l2, l3, map_pad, w_vec)
    return jnp.sum(partials)


def kernel(level2_pred, level3_pred, mapping, weight):
    return _loss(level2_pred, level3_pred, mapping,
                 jnp.asarray(weight, jnp.float32))


# speedup vs baseline: 1.1627x; 1.1627x over previous
import functools

import jax
import jax.numpy as jnp
from jax import lax
from jax.experimental import pallas as pl
from jax.experimental.pallas import tpu as pltpu
from jax.experimental.pallas import tpu_sc as plsc

B = 4
C3 = 30
C2 = 10
H = 512
W = 512
HW = H * W

NC = 2
NS = 16
L = 16
NW = NC * NS

PIX_PER_W = (B * HW) // NW
CHUNK = 1024
CHUNKS_PER_B = (HW // NW) // CHUNK
NT = B * CHUNKS_PER_B

_mesh = plsc.VectorSubcoreMesh(core_axis_name="c", subcore_axis_name="s")


@functools.partial(
    pl.kernel,
    mesh=_mesh,
    out_type=jax.ShapeDtypeStruct((NW, L), jnp.float32),
    scratch_types=[
        pltpu.VMEM((C3, CHUNK), jnp.float32),
        pltpu.VMEM((C3, CHUNK), jnp.float32),
        pltpu.VMEM((C2, CHUNK), jnp.float32),
        pltpu.VMEM((C2, CHUNK), jnp.float32),
        pltpu.VMEM((32,), jnp.int32),
        pltpu.VMEM((L,), jnp.float32),
        pltpu.VMEM((L,), jnp.float32),
        pltpu.SemaphoreType.DMA,
        pltpu.SemaphoreType.DMA,
    ],
)
def _sc_loss(l2_hbm, l3_hbm, map_hbm, w_hbm, out_hbm,
             l3a, l3b, l2a, l2b, mapv, wv, ov, sema, semb):
    wid = lax.axis_index("s") * NC + lax.axis_index("c")

    pltpu.sync_copy(map_hbm, mapv)
    pltpu.sync_copy(w_hbm, wv)

    mv0 = mapv[pl.ds(0, L)]
    mv1 = mapv[pl.ds(L, L)]
    mks = [mv0[k] if k < L else mv1[k - L] for k in range(C3)]

    l3bufs = (l3a, l3b)
    l2bufs = (l2a, l2b)
    sems = (sema, semb)

    def start(t):
        i = t % 2
        b = t // CHUNKS_PER_B
        c = t % CHUNKS_PER_B
        off = wid * (HW // NW) + c * CHUNK
        h3 = pltpu.async_copy(l3_hbm.at[b, :, pl.ds(off, CHUNK)],
                              l3bufs[i], sems[i])
        h2 = pltpu.async_copy(l2_hbm.at[b, :, pl.ds(off, CHUNK)],
                              l2bufs[i], sems[i])
        return h3, h2

    def compute(t, acc):
        i = t % 2
        l3v, l2v = l3bufs[i], l2bufs[i]

        def group(j, acc_in):
            s = pl.ds(j * L, L)
            m3 = l3v[0, s]
            mapped = jnp.full((L,), mks[0], jnp.int32)
            for k in range(1, C3):
                v = l3v[k, s]
                upd = v > m3
                m3 = jnp.where(upd, v, m3)
                mapped = jnp.where(upd, mks[k], mapped)
            m2 = l2v[0, s]
            i2 = jnp.zeros((L,), jnp.int32)
            for k in range(1, C2):
                v = l2v[k, s]
                upd = v > m2
                m2 = jnp.where(upd, v, m2)
                i2 = jnp.where(upd, k, i2)
            return acc_in + jnp.where(mapped != i2, 1.0, 0.0)

        return lax.fori_loop(0, CHUNK // L, group, acc)

    acc = jnp.zeros((L,), jnp.float32)
    pending = start(0)
    for t in range(NT):
        nxt = start(t + 1) if t + 1 < NT else ()
        for h in pending:
            h.wait()
        acc = compute(t, acc)
        pending = nxt

    ov[...] = acc * wv[...] * (1.0 / (B * HW))
    pltpu.sync_copy(ov, out_hbm.at[wid])


@jax.jit
def _loss(level2_pred, level3_pred, mapping, weight):
    l2 = level2_pred.reshape(B, C2, HW)
    l3 = level3_pred.reshape(B, C3, HW)
    map_pad = jnp.pad(mapping, (0, 32 - C3))
    w_vec = jnp.broadcast_to(jnp.asarray(weight, jnp.float32), (L,))
    partials = _sc_loss(l2, l3, map_pad, w_vec)
    return jnp.sum(partials)


def kernel(level2_pred, level3_pred, mapping, weight):
    return _loss(level2_pred, level3_pred, mapping,
                 jnp.asarray(weight, jnp.float32))
